# Initial kernel scaffold; baseline (speedup 1.0000x reference)
import jax
import jax.numpy as jnp
from jax.experimental import pallas as pl


def kernel(cost_vals, a, b, coords, epsilon, lmbda, cost_rows, cost_cols):
    raise NotImplementedError("write your pallas kernel here")


# speedup vs baseline: 203.3116x; 203.3116x over previous
import functools

import jax
import jax.numpy as jnp
from jax import lax
from jax.experimental import pallas as pl
from jax.experimental.pallas import tpu as pltpu
from jax.experimental.pallas import tpu_sc as plsc

N = 65536
NC = 2
NS = 16
NW = NC * NS
L = 16
CHUNK = 4096
GROUPS = CHUNK // L
SROWS = CHUNK // 128
TSLICE = N // NS

N_CHUNKS = 33
E_PAD = NW * N_CHUNKS * CHUNK

_MESH = plsc.VectorSubcoreMesh(core_axis_name="c", subcore_axis_name="s",
                               num_cores=NC, num_subcores=NS)


def _zero_fill(buf, nwords):
    def body(i, carry):
        buf[pl.ds(i * L, L)] = jnp.zeros((L,), jnp.float32)
        return carry
    lax.fori_loop(0, nwords // L, body, 0)


def _spmv_kernel(use_exp: bool, write_prod: bool, interpret=False):
    out_type = [jax.ShapeDtypeStruct((NC, N), jnp.float32)]
    if write_prod:
        out_type.append(jax.ShapeDtypeStruct((E_PAD,), jnp.float32))

    def body(vals_hbm, gidx_hbm, sidx_hbm, x_hbm, *refs):
        if write_prod:
            out_hbm, prod_hbm, xloc, zbuf, cvals, cgidx, csidx, prodb, acc_sh = refs
        else:
            out_hbm, xloc, zbuf, cvals, cgidx, csidx, prodb, acc_sh = refs
        cid = lax.axis_index("c")
        sid = lax.axis_index("s")
        wid = cid * NS + sid
        pltpu.sync_copy(x_hbm, xloc)
        _zero_fill(zbuf, TSLICE)
        pltpu.sync_copy(zbuf, acc_sh.at[pl.ds(sid * TSLICE, TSLICE)])
        plsc.subcore_barrier()

        base = wid * (N_CHUNKS * CHUNK)
        row_base = wid * (N_CHUNKS * SROWS)

        def chunk_body(c, carry):
            off = base + c * CHUNK
            pltpu.sync_copy(vals_hbm.at[pl.ds(off, CHUNK)], cvals)
            pltpu.sync_copy(gidx_hbm.at[pl.ds(off, CHUNK)], cgidx)
            pltpu.sync_copy(sidx_hbm.at[pl.ds(row_base + c * SROWS, SROWS)],
                            csidx)

            def grp(g, carry2):
                k = cvals[pl.ds(g * L, L)]
                if use_exp:
                    k = jnp.exp(k)
                idx = cgidx[pl.ds(g * L, L)]
                gathered = plsc.load_gather(xloc, [idx])
                prodb[pl.ds(g * L, L)] = k * gathered
                return carry2
            lax.fori_loop(0, GROUPS, grp, 0)

            if write_prod:
                pltpu.sync_copy(prodb, prod_hbm.at[pl.ds(off, CHUNK)])

            def srow(j, carry2):
                pltpu.sync_copy(prodb.at[pl.ds(j * 128, 128)],
                                acc_sh.at[csidx.at[j]], add=True)
                return carry2
            lax.fori_loop(0, SROWS, srow, 0)
            return carry
        lax.fori_loop(0, N_CHUNKS, chunk_body, 0)

        plsc.subcore_barrier()
        pltpu.sync_copy(acc_sh.at[pl.ds(sid * TSLICE, TSLICE)],
                        out_hbm.at[cid, pl.ds(sid * TSLICE, TSLICE)])

    return pl.kernel(
        body,
        out_type=tuple(out_type) if write_prod else out_type[0],
        mesh=_MESH,
        scratch_types=[
            pltpu.VMEM((N,), jnp.float32),
            pltpu.VMEM((TSLICE,), jnp.float32),
            pltpu.VMEM((CHUNK,), jnp.float32),
            pltpu.VMEM((CHUNK,), jnp.int32),
            pltpu.VMEM((SROWS, 128), jnp.int32),
            pltpu.VMEM((CHUNK,), jnp.float32),
            pltpu.VMEM_SHARED((N,), jnp.float32),
        ],
        compiler_params=pltpu.CompilerParams(needs_layout_passes=False),
        interpret=interpret,
    )


def _proj_kernel(interpret=False):
    n_chunks16 = E_PAD // (NS * CHUNK)

    def body(t_hbm, gidx_hbm, sidx_hbm, cx_hbm, cy_hbm, den_hbm, out_hbm,
             xloc, zbuf, cvals, cgidx, csidx, prodb, d0, d1, acc_sh):
        cid = lax.axis_index("c")
        sid = lax.axis_index("s")

        @pl.when(cid == 0)
        def _():
            pltpu.sync_copy(cx_hbm, xloc)

        @pl.when(cid == 1)
        def _():
            pltpu.sync_copy(cy_hbm, xloc)

        _zero_fill(zbuf, TSLICE)
        pltpu.sync_copy(zbuf, acc_sh.at[pl.ds(sid * TSLICE, TSLICE)])
        plsc.subcore_barrier()

        base = sid * (n_chunks16 * CHUNK)
        row_base = sid * (n_chunks16 * SROWS)

        def chunk_body(c, carry):
            off = base + c * CHUNK
            pltpu.sync_copy(t_hbm.at[pl.ds(off, CHUNK)], cvals)
            pltpu.sync_copy(gidx_hbm.at[pl.ds(off, CHUNK)], cgidx)
            pltpu.sync_copy(sidx_hbm.at[pl.ds(row_base + c * SROWS, SROWS)],
                            csidx)

            def grp(g, carry2):
                t = cvals[pl.ds(g * L, L)]
                idx = cgidx[pl.ds(g * L, L)]
                coord = plsc.load_gather(xloc, [idx])
                prodb[pl.ds(g * L, L)] = t * coord
                return carry2
            lax.fori_loop(0, GROUPS, grp, 0)

            def srow(j, carry2):
                pltpu.sync_copy(prodb.at[pl.ds(j * 128, 128)],
                                acc_sh.at[csidx.at[j]], add=True)
                return carry2
            lax.fori_loop(0, SROWS, srow, 0)
            return carry
        lax.fori_loop(0, n_chunks16, chunk_body, 0)

        plsc.subcore_barrier()
        sl = pl.ds(sid * TSLICE, TSLICE)
        pltpu.sync_copy(den_hbm.at[0, sl], d0)
        pltpu.sync_copy(den_hbm.at[1, sl], d1)
        pltpu.sync_copy(acc_sh.at[sl], zbuf)

        def divg(g, carry):
            s = pl.ds(g * L, L)
            zbuf[s] = zbuf[s] / (d0[s] + d1[s])
            return carry
        lax.fori_loop(0, TSLICE // L, divg, 0)
        pltpu.sync_copy(zbuf, out_hbm.at[cid, sl])

    return pl.kernel(
        body,
        out_type=jax.ShapeDtypeStruct((NC, N), jnp.float32),
        mesh=_MESH,
        scratch_types=[
            pltpu.VMEM((N,), jnp.float32),
            pltpu.VMEM((TSLICE,), jnp.float32),
            pltpu.VMEM((CHUNK,), jnp.float32),
            pltpu.VMEM((CHUNK,), jnp.int32),
            pltpu.VMEM((SROWS, 128), jnp.int32),
            pltpu.VMEM((CHUNK,), jnp.float32),
            pltpu.VMEM((TSLICE,), jnp.float32),
            pltpu.VMEM((TSLICE,), jnp.float32),
            pltpu.VMEM_SHARED((N,), jnp.float32),
        ],
        compiler_params=pltpu.CompilerParams(needs_layout_passes=False),
        interpret=interpret,
    )


def _pow_body(p_ref, ab_ref, kvp_ref, o_ref):
    base = ab_ref[...] / (kvp_ref[0] + kvp_ref[1])
    o_ref[...] = jnp.power(base, p_ref[0])


def _tc_pow(p, ab, kvp):
    out = pl.pallas_call(
        _pow_body,
        out_shape=jax.ShapeDtypeStruct((N // 128, 128), jnp.float32),
        in_specs=[
            pl.BlockSpec(memory_space=pltpu.SMEM),
            pl.BlockSpec(memory_space=pltpu.VMEM),
            pl.BlockSpec(memory_space=pltpu.VMEM),
        ],
        out_specs=pl.BlockSpec(memory_space=pltpu.VMEM),
    )(p.reshape(1), ab.reshape(N // 128, 128),
      kvp.reshape(NC, N // 128, 128))
    return out.reshape(N)


def kernel(cost_vals, a, b, coords, epsilon, lmbda, cost_rows, cost_cols):
    p = (lmbda / (lmbda + epsilon)).astype(jnp.float32)
    neg_cost = cost_vals * (-1.0 / epsilon)

    pad = E_PAD - cost_vals.shape[0]
    neg_cost = jnp.pad(neg_cost, (0, pad), constant_values=-1e30)
    rows1 = jnp.pad(cost_rows, (0, pad))
    cols1 = jnp.pad(cost_cols, (0, pad))
    rows2 = rows1.reshape(E_PAD // 128, 128)
    cols2 = cols1.reshape(E_PAD // 128, 128)

    spmv = _spmv_kernel(use_exp=True, write_prod=False)
    spmv_p = _spmv_kernel(use_exp=True, write_prod=True)
    spmv_t = _spmv_kernel(use_exp=False, write_prod=True)
    proj = _proj_kernel()

    v = jnp.full((N,), 1.0 / N, jnp.float32)
    u = None
    for it in range(10):
        kvp = spmv(neg_cost, cols1, rows2, v)
        u = _tc_pow(p, a, kvp)
        if it < 9:
            kup = spmv(neg_cost, rows1, cols2, u)
        else:
            kup, g = spmv_p(neg_cost, rows1, cols2, u)
        v = _tc_pow(p, b, kup)

    denp, t = spmv_t(g, cols1, rows2, v)
    out2 = proj(t, cols1, rows2, coords[:, 0], coords[:, 1], denp)
    return out2.T


# speedup vs baseline: 387.8057x; 1.9074x over previous
import functools

import jax
import jax.numpy as jnp
from jax import lax
from jax.experimental import pallas as pl
from jax.experimental.pallas import tpu as pltpu
from jax.experimental.pallas import tpu_sc as plsc

N = 65536
NC = 2
NS = 16
NW = NC * NS
L = 16
CHUNK = 4096
GROUPS = CHUNK // L
SROWS = CHUNK // 128
TSLICE = N // NS

N_CHUNKS = 33
E_PAD = NW * N_CHUNKS * CHUNK
NBUF = 3

_MESH = plsc.VectorSubcoreMesh(core_axis_name="c", subcore_axis_name="s",
                               num_cores=NC, num_subcores=NS)


def _zero_fill(buf, nwords):
    def body(i, carry):
        buf[pl.ds(i * L, L)] = jnp.zeros((L,), jnp.float32)
        return carry
    lax.fori_loop(0, nwords // L, body, 0)


def _spmv_kernel(use_exp: bool, write_prod: bool, interpret=False):
    out_type = [jax.ShapeDtypeStruct((NC, N), jnp.float32)]
    if write_prod:
        out_type.append(jax.ShapeDtypeStruct((E_PAD,), jnp.float32))

    def body(vals_hbm, gidx_hbm, sidx_hbm, x_hbm, *refs):
        if write_prod:
            out_hbm, prod_hbm = refs[0], refs[1]
            refs = refs[2:]
        else:
            out_hbm = refs[0]
            refs = refs[1:]
        xloc, zbuf = refs[0], refs[1]
        cvals = refs[2:2 + NBUF]
        cgidx = refs[2 + NBUF:2 + 2 * NBUF]
        csidx = refs[2 + 2 * NBUF:2 + 3 * NBUF]
        prodb = refs[2 + 3 * NBUF:2 + 4 * NBUF]
        acc_sh = refs[2 + 4 * NBUF]
        lsem = refs[3 + 4 * NBUF:3 + 5 * NBUF]
        ssem = refs[3 + 5 * NBUF:3 + 6 * NBUF]

        cid = lax.axis_index("c")
        sid = lax.axis_index("s")
        wid = cid * NS + sid
        base = wid * (N_CHUNKS * CHUNK)
        row_base = wid * (N_CHUNKS * SROWS)

        def start_loads(c, bi):
            pltpu.async_copy(vals_hbm.at[pl.ds(base + c * CHUNK, CHUNK)],
                             cvals[bi], lsem[bi])
            pltpu.async_copy(gidx_hbm.at[pl.ds(base + c * CHUNK, CHUNK)],
                             cgidx[bi], lsem[bi])
            pltpu.async_copy(sidx_hbm.at[pl.ds(base + c * CHUNK, CHUNK)],
                             csidx[bi], lsem[bi])

        def wait_loads(bi):
            pltpu.make_async_copy(vals_hbm.at[pl.ds(0, CHUNK)],
                                  cvals[bi], lsem[bi]).wait()
            pltpu.make_async_copy(gidx_hbm.at[pl.ds(0, CHUNK)],
                                  cgidx[bi], lsem[bi]).wait()
            pltpu.make_async_copy(sidx_hbm.at[pl.ds(0, CHUNK)],
                                  csidx[bi], lsem[bi]).wait()

        def wait_scatter(bi):
            pltpu.make_async_copy(prodb[bi], acc_sh.at[csidx[bi]],
                                  ssem[bi]).wait()

        pltpu.sync_copy(x_hbm, xloc)
        _zero_fill(zbuf, TSLICE)
        pltpu.sync_copy(zbuf, acc_sh.at[pl.ds(sid * TSLICE, TSLICE)])
        plsc.subcore_barrier()

        start_loads(0, 0)

        def compute(bi):
            def row_body(r, carry):
                for j in range(8):
                    o = r * 128 + j * L
                    k = cvals[bi][pl.ds(o, L)]
                    if use_exp:
                        k = jnp.exp(k)
                    idx = cgidx[bi][pl.ds(o, L)]
                    prodb[bi][pl.ds(o, L)] = (
                        k * plsc.load_gather(xloc, [idx]))
                return carry
            lax.fori_loop(0, SROWS, row_body, 0)

        def ring_body(h, carry):
            for p in range(NBUF):
                bi = p
                c = h * NBUF + p
                wait_loads(bi)
                compute(bi)
                if write_prod:
                    pltpu.sync_copy(
                        prodb[bi],
                        prod_hbm.at[pl.ds(base + c * CHUNK, CHUNK)])
                pltpu.async_copy(prodb[bi], acc_sh.at[csidx[bi]],
                                 ssem[bi], add=True)
                nbi = (p + 1) % NBUF
                if p == NBUF - 1:
                    wait_scatter(nbi)
                    start_loads(jnp.minimum(c + 1, N_CHUNKS - 1), nbi)
                else:
                    @pl.when(h >= 1)
                    def _():
                        wait_scatter(nbi)
                    start_loads(c + 1, nbi)
            return carry
        lax.fori_loop(0, N_CHUNKS // NBUF, ring_body, 0)

        wait_loads(0)
        wait_scatter(1)
        wait_scatter(2)

        plsc.subcore_barrier()
        pltpu.sync_copy(acc_sh.at[pl.ds(sid * TSLICE, TSLICE)],
                        out_hbm.at[cid, pl.ds(sid * TSLICE, TSLICE)])

    scratch = [
        pltpu.VMEM((N,), jnp.float32),
        pltpu.VMEM((TSLICE,), jnp.float32),
    ]
    scratch += [pltpu.VMEM((CHUNK,), jnp.float32) for _ in range(NBUF)]
    scratch += [pltpu.VMEM((CHUNK,), jnp.int32) for _ in range(NBUF)]
    scratch += [pltpu.VMEM((CHUNK,), jnp.int32) for _ in range(NBUF)]
    scratch += [pltpu.VMEM((CHUNK,), jnp.float32) for _ in range(NBUF)]
    scratch += [pltpu.VMEM_SHARED((N,), jnp.float32)]
    scratch += [pltpu.SemaphoreType.DMA for _ in range(2 * NBUF)]

    return pl.kernel(
        body,
        out_type=tuple(out_type) if write_prod else out_type[0],
        mesh=_MESH,
        scratch_types=scratch,
        compiler_params=pltpu.CompilerParams(needs_layout_passes=False),
        interpret=interpret,
    )


def _proj_kernel(interpret=False):
    n_chunks16 = E_PAD // (NS * CHUNK)

    def body(t_hbm, gidx_hbm, sidx_hbm, cx_hbm, cy_hbm, den_hbm, out_hbm,
             xloc, zbuf, cvals, cgidx, csidx, prodb, d0, d1, acc_sh):
        cid = lax.axis_index("c")
        sid = lax.axis_index("s")

        @pl.when(cid == 0)
        def _():
            pltpu.sync_copy(cx_hbm, xloc)

        @pl.when(cid == 1)
        def _():
            pltpu.sync_copy(cy_hbm, xloc)

        _zero_fill(zbuf, TSLICE)
        pltpu.sync_copy(zbuf, acc_sh.at[pl.ds(sid * TSLICE, TSLICE)])
        plsc.subcore_barrier()

        base = sid * (n_chunks16 * CHUNK)
        row_base = sid * (n_chunks16 * SROWS)

        def chunk_body(c, carry):
            off = base + c * CHUNK
            pltpu.sync_copy(t_hbm.at[pl.ds(off, CHUNK)], cvals)
            pltpu.sync_copy(gidx_hbm.at[pl.ds(off, CHUNK)], cgidx)
            pltpu.sync_copy(sidx_hbm.at[pl.ds(row_base + c * SROWS, SROWS)],
                            csidx)

            def grp(g, carry2):
                t = cvals[pl.ds(g * L, L)]
                idx = cgidx[pl.ds(g * L, L)]
                coord = plsc.load_gather(xloc, [idx])
                prodb[pl.ds(g * L, L)] = t * coord
                return carry2
            lax.fori_loop(0, GROUPS, grp, 0)

            def srow(j, carry2):
                pltpu.sync_copy(prodb.at[pl.ds(j * 128, 128)],
                                acc_sh.at[csidx.at[j]], add=True)
                return carry2
            lax.fori_loop(0, SROWS, srow, 0)
            return carry
        lax.fori_loop(0, n_chunks16, chunk_body, 0)

        plsc.subcore_barrier()
        sl = pl.ds(sid * TSLICE, TSLICE)
        pltpu.sync_copy(den_hbm.at[0, sl], d0)
        pltpu.sync_copy(den_hbm.at[1, sl], d1)
        pltpu.sync_copy(acc_sh.at[sl], zbuf)

        def divg(g, carry):
            s = pl.ds(g * L, L)
            zbuf[s] = zbuf[s] / (d0[s] + d1[s])
            return carry
        lax.fori_loop(0, TSLICE // L, divg, 0)
        pltpu.sync_copy(zbuf, out_hbm.at[cid, sl])

    return pl.kernel(
        body,
        out_type=jax.ShapeDtypeStruct((NC, N), jnp.float32),
        mesh=_MESH,
        scratch_types=[
            pltpu.VMEM((N,), jnp.float32),
            pltpu.VMEM((TSLICE,), jnp.float32),
            pltpu.VMEM((CHUNK,), jnp.float32),
            pltpu.VMEM((CHUNK,), jnp.int32),
            pltpu.VMEM((SROWS, 128), jnp.int32),
            pltpu.VMEM((CHUNK,), jnp.float32),
            pltpu.VMEM((TSLICE,), jnp.float32),
            pltpu.VMEM((TSLICE,), jnp.float32),
            pltpu.VMEM_SHARED((N,), jnp.float32),
        ],
        compiler_params=pltpu.CompilerParams(needs_layout_passes=False),
        interpret=interpret,
    )


def _pow_body(p_ref, ab_ref, kvp_ref, o_ref):
    base = ab_ref[...] / (kvp_ref[0] + kvp_ref[1])
    o_ref[...] = jnp.power(base, p_ref[0])


def _tc_pow(p, ab, kvp):
    out = pl.pallas_call(
        _pow_body,
        out_shape=jax.ShapeDtypeStruct((N // 128, 128), jnp.float32),
        in_specs=[
            pl.BlockSpec(memory_space=pltpu.SMEM),
            pl.BlockSpec(memory_space=pltpu.VMEM),
            pl.BlockSpec(memory_space=pltpu.VMEM),
        ],
        out_specs=pl.BlockSpec(memory_space=pltpu.VMEM),
    )(p.reshape(1), ab.reshape(N // 128, 128),
      kvp.reshape(NC, N // 128, 128))
    return out.reshape(N)


def kernel(cost_vals, a, b, coords, epsilon, lmbda, cost_rows, cost_cols):
    p = (lmbda / (lmbda + epsilon)).astype(jnp.float32)
    neg_cost = cost_vals * (-1.0 / epsilon)

    pad = E_PAD - cost_vals.shape[0]
    neg_cost = jnp.pad(neg_cost, (0, pad), constant_values=-1e30)
    rows1 = jnp.pad(cost_rows, (0, pad))
    cols1 = jnp.pad(cost_cols, (0, pad))
    rows2 = rows1.reshape(E_PAD // 128, 128)

    spmv = _spmv_kernel(use_exp=True, write_prod=False)
    spmv_p = _spmv_kernel(use_exp=True, write_prod=True)
    spmv_t = _spmv_kernel(use_exp=False, write_prod=True)
    proj = _proj_kernel()

    v = jnp.full((N,), 1.0 / N, jnp.float32)
    u = None
    for it in range(10):
        kvp = spmv(neg_cost, cols1, rows1, v)
        u = _tc_pow(p, a, kvp)
        if it < 9:
            kup = spmv(neg_cost, rows1, cols1, u)
        else:
            kup, g = spmv_p(neg_cost, rows1, cols1, u)
        v = _tc_pow(p, b, kup)

    denp, t = spmv_t(g, cols1, rows1, v)
    out2 = proj(t, cols1, rows2, coords[:, 0], coords[:, 1], denp)
    return out2.T


# speedup vs baseline: 484.9422x; 1.2505x over previous
import jax
import jax.numpy as jnp
from jax import lax
from jax.experimental import pallas as pl
from jax.experimental.pallas import tpu as pltpu
from jax.experimental.pallas import tpu_sc as plsc

N = 65536
NC = 2
NS = 16
NW = NC * NS
L = 16
CHUNK = 4224
GROUPS = CHUNK // L
TSLICE = N // NS
N_CHUNKS = 32
E_PAD = NW * N_CHUNKS * CHUNK
NBUF = 4
UNROLL = 8

_MESH = plsc.VectorSubcoreMesh(core_axis_name="c", subcore_axis_name="s",
                               num_cores=NC, num_subcores=NS)
_PARAMS = pltpu.CompilerParams(needs_layout_passes=False)


def _zero_fill(buf, nwords):
    def body(i, carry):
        for j in range(UNROLL):
            buf[pl.ds((i * UNROLL + j) * L, L)] = jnp.zeros((L,), jnp.float32)
        return carry
    lax.fori_loop(0, nwords // (L * UNROLL), body, 0)


def _ring_pass(nchunks, base, cvals, cgidx, csidx, lsem, ssem, acc_sh,
               vals_hbm, gidx_hbm, sidx_hbm, compute_chunk, after_compute):
    def start_loads(c, bi):
        off = base + c * CHUNK
        pltpu.async_copy(vals_hbm.at[pl.ds(off, CHUNK)], cvals[bi], lsem[bi])
        pltpu.async_copy(gidx_hbm.at[pl.ds(off, CHUNK)], cgidx[bi], lsem[bi])
        pltpu.async_copy(sidx_hbm.at[pl.ds(off, CHUNK)], csidx[bi], lsem[bi])

    def wait_loads(bi):
        pltpu.make_async_copy(vals_hbm.at[pl.ds(0, CHUNK)],
                              cvals[bi], lsem[bi]).wait()
        pltpu.make_async_copy(gidx_hbm.at[pl.ds(0, CHUNK)],
                              cgidx[bi], lsem[bi]).wait()
        pltpu.make_async_copy(sidx_hbm.at[pl.ds(0, CHUNK)],
                              csidx[bi], lsem[bi]).wait()

    def wait_scatter(bi):
        pltpu.make_async_copy(cvals[bi], acc_sh.at[csidx[bi]],
                              ssem[bi]).wait()

    start_loads(0, 0)
    start_loads(1, 1)

    def ring_body(h, carry):
        for p in range(NBUF):
            bi = p
            c = h * NBUF + p
            wait_loads(bi)
            compute_chunk(bi)
            after_compute(c, bi)
            pltpu.async_copy(cvals[bi], acc_sh.at[csidx[bi]],
                             ssem[bi], add=True)
            nbi = (p + 2) % NBUF
            if p >= 2:
                wait_scatter(nbi)
                start_loads(jnp.minimum(c + 2, nchunks - 1), nbi)
            else:
                @pl.when(h >= 1)
                def _():
                    wait_scatter(nbi)
                start_loads(c + 2, nbi)
        return carry
    lax.fori_loop(0, nchunks // NBUF, ring_body, 0)

    wait_loads(0)
    wait_loads(1)
    wait_scatter(2)
    wait_scatter(3)


def _spmv_kernel(use_exp: bool, write_prod: bool, interpret=False):
    out_type = [jax.ShapeDtypeStruct((NC, N), jnp.float32)]
    if write_prod:
        out_type.append(jax.ShapeDtypeStruct((E_PAD,), jnp.float32))

    def body(vals_hbm, gidx_hbm, sidx_hbm, x_hbm, *refs):
        if write_prod:
            out_hbm, prod_hbm = refs[0], refs[1]
            refs = refs[2:]
        else:
            out_hbm = refs[0]
            refs = refs[1:]
        xloc = refs[0]
        cvals = refs[1:1 + NBUF]
        cgidx = refs[1 + NBUF:1 + 2 * NBUF]
        csidx = refs[1 + 2 * NBUF:1 + 3 * NBUF]
        acc_sh = refs[1 + 3 * NBUF]
        lsem = refs[2 + 3 * NBUF:2 + 4 * NBUF]
        ssem = refs[2 + 4 * NBUF:2 + 5 * NBUF]
        xsem = refs[2 + 5 * NBUF]
        zbuf = cvals[2]

        cid = lax.axis_index("c")
        sid = lax.axis_index("s")
        wid = cid * NS + sid
        base = wid * (N_CHUNKS * CHUNK)

        xcopy = pltpu.async_copy(x_hbm, xloc, xsem)
        _zero_fill(zbuf, TSLICE)
        pltpu.sync_copy(zbuf.at[pl.ds(0, TSLICE)],
                        acc_sh.at[pl.ds(sid * TSLICE, TSLICE)])
        xcopy.wait()
        plsc.subcore_barrier()

        def compute_chunk(bi):
            def row_body(r, carry):
                for j in range(UNROLL):
                    o = (r * UNROLL + j) * L
                    k = cvals[bi][pl.ds(o, L)]
                    if use_exp:
                        k = jnp.exp(k)
                    idx = cgidx[bi][pl.ds(o, L)]
                    cvals[bi][pl.ds(o, L)] = k * plsc.load_gather(xloc, [idx])
                return carry
            lax.fori_loop(0, GROUPS // UNROLL, row_body, 0)

        def after_compute(c, bi):
            if write_prod:
                pltpu.sync_copy(cvals[bi],
                                prod_hbm.at[pl.ds(base + c * CHUNK, CHUNK)])

        _ring_pass(N_CHUNKS, base, cvals, cgidx, csidx, lsem, ssem, acc_sh,
                   vals_hbm, gidx_hbm, sidx_hbm, compute_chunk, after_compute)

        plsc.subcore_barrier()
        pltpu.sync_copy(acc_sh.at[pl.ds(sid * TSLICE, TSLICE)],
                        out_hbm.at[cid, pl.ds(sid * TSLICE, TSLICE)])

    scratch = [
        pltpu.VMEM((N,), jnp.float32),
    ]
    scratch += [pltpu.VMEM((CHUNK,), jnp.float32) for _ in range(NBUF)]
    scratch += [pltpu.VMEM((CHUNK,), jnp.int32) for _ in range(NBUF)]
    scratch += [pltpu.VMEM((CHUNK,), jnp.int32) for _ in range(NBUF)]
    scratch += [pltpu.VMEM_SHARED((N,), jnp.float32)]
    scratch += [pltpu.SemaphoreType.DMA for _ in range(2 * NBUF + 1)]

    return pl.kernel(
        body,
        out_type=tuple(out_type) if write_prod else out_type[0],
        mesh=_MESH,
        scratch_types=scratch,
        compiler_params=_PARAMS,
        interpret=interpret,
    )


def _proj_kernel(interpret=False):
    n_chunks16 = E_PAD // (NS * CHUNK)

    def body(t_hbm, gidx_hbm, sidx_hbm, cx_hbm, cy_hbm, den_hbm, out_hbm,
             *refs):
        xloc = refs[0]
        cvals = refs[1:1 + NBUF]
        cgidx = refs[1 + NBUF:1 + 2 * NBUF]
        csidx = refs[1 + 2 * NBUF:1 + 3 * NBUF]
        acc_sh = refs[1 + 3 * NBUF]
        lsem = refs[2 + 3 * NBUF:2 + 4 * NBUF]
        ssem = refs[2 + 4 * NBUF:2 + 5 * NBUF]
        xsem = refs[2 + 5 * NBUF]
        zbuf = cvals[2]

        cid = lax.axis_index("c")
        sid = lax.axis_index("s")
        base = sid * (n_chunks16 * CHUNK)

        @pl.when(cid == 0)
        def _():
            pltpu.async_copy(cx_hbm, xloc, xsem)

        @pl.when(cid == 1)
        def _():
            pltpu.async_copy(cy_hbm, xloc, xsem)

        _zero_fill(zbuf, TSLICE)
        pltpu.sync_copy(zbuf.at[pl.ds(0, TSLICE)],
                        acc_sh.at[pl.ds(sid * TSLICE, TSLICE)])
        pltpu.make_async_copy(cx_hbm, xloc, xsem).wait()
        plsc.subcore_barrier()

        def compute_chunk(bi):
            def row_body(r, carry):
                for j in range(UNROLL):
                    o = (r * UNROLL + j) * L
                    t = cvals[bi][pl.ds(o, L)]
                    idx = cgidx[bi][pl.ds(o, L)]
                    cvals[bi][pl.ds(o, L)] = t * plsc.load_gather(xloc, [idx])
                return carry
            lax.fori_loop(0, GROUPS // UNROLL, row_body, 0)

        def after_compute(c, bi):
            pass

        _ring_pass(n_chunks16, base, cvals, cgidx, csidx, lsem, ssem, acc_sh,
                   t_hbm, gidx_hbm, sidx_hbm, compute_chunk, after_compute)

        plsc.subcore_barrier()
        d0, d1 = cvals[0], cvals[1]
        sl = pl.ds(sid * TSLICE, TSLICE)
        pltpu.sync_copy(den_hbm.at[0, sl], d0.at[pl.ds(0, TSLICE)])
        pltpu.sync_copy(den_hbm.at[1, sl], d1.at[pl.ds(0, TSLICE)])
        pltpu.sync_copy(acc_sh.at[sl], zbuf.at[pl.ds(0, TSLICE)])

        def divg(g, carry):
            s = pl.ds(g * L, L)
            zbuf[s] = zbuf[s] / (d0[s] + d1[s])
            return carry
        lax.fori_loop(0, TSLICE // L, divg, 0)
        pltpu.sync_copy(zbuf.at[pl.ds(0, TSLICE)], out_hbm.at[cid, sl])

    scratch = [
        pltpu.VMEM((N,), jnp.float32),
    ]
    scratch += [pltpu.VMEM((CHUNK,), jnp.float32) for _ in range(NBUF)]
    scratch += [pltpu.VMEM((CHUNK,), jnp.int32) for _ in range(NBUF)]
    scratch += [pltpu.VMEM((CHUNK,), jnp.int32) for _ in range(NBUF)]
    scratch += [pltpu.VMEM_SHARED((N,), jnp.float32)]
    scratch += [pltpu.SemaphoreType.DMA for _ in range(2 * NBUF + 1)]

    return pl.kernel(
        body,
        out_type=jax.ShapeDtypeStruct((NC, N), jnp.float32),
        mesh=_MESH,
        scratch_types=scratch,
        compiler_params=_PARAMS,
        interpret=interpret,
    )


def _pow_body(p_ref, ab_ref, kvp_ref, o_ref):
    base = ab_ref[...] / (kvp_ref[0] + kvp_ref[1])
    o_ref[...] = jnp.power(base, p_ref[0])


def _tc_pow(p, ab, kvp):
    out = pl.pallas_call(
        _pow_body,
        out_shape=jax.ShapeDtypeStruct((N // 128, 128), jnp.float32),
        in_specs=[
            pl.BlockSpec(memory_space=pltpu.SMEM),
            pl.BlockSpec(memory_space=pltpu.VMEM),
            pl.BlockSpec(memory_space=pltpu.VMEM),
        ],
        out_specs=pl.BlockSpec(memory_space=pltpu.VMEM),
    )(p.reshape(1), ab.reshape(N // 128, 128),
      kvp.reshape(NC, N // 128, 128))
    return out.reshape(N)


def kernel(cost_vals, a, b, coords, epsilon, lmbda, cost_rows, cost_cols):
    p = (lmbda / (lmbda + epsilon)).astype(jnp.float32)
    neg_cost = cost_vals * (-1.0 / epsilon)

    pad = E_PAD - cost_vals.shape[0]
    neg_cost = jnp.pad(neg_cost, (0, pad), constant_values=-1e30)
    rows1 = jnp.pad(cost_rows, (0, pad))
    cols1 = jnp.pad(cost_cols, (0, pad))
    a_s = a * (1.0 / N)
    b_s = b * (1.0 / N)

    spmv1 = _spmv_kernel(use_exp=True, write_prod=True)
    spmv = _spmv_kernel(use_exp=False, write_prod=False)
    spmv_p = _spmv_kernel(use_exp=False, write_prod=True)
    proj = _proj_kernel()

    v0 = jnp.full((N,), 1.0 / N, jnp.float32)
    kvp, kn = spmv1(neg_cost, cols1, rows1, v0)
    u = _tc_pow(p, a, kvp)
    v = None
    for it in range(10):
        if it > 0:
            kvp = spmv(kn, cols1, rows1, v)
            u = _tc_pow(p, a_s, kvp)
        if it < 9:
            kup = spmv(kn, rows1, cols1, u)
        else:
            kup, g = spmv_p(kn, rows1, cols1, u)
        v = _tc_pow(p, b_s, kup)

    denp, t = spmv_p(g, cols1, rows1, v)
    out2 = proj(t, cols1, rows1, coords[:, 0], coords[:, 1], denp)
    return out2.T
